# baseline (speedup 1.0000x reference)
import functools

import jax
import jax.numpy as jnp
from jax import lax
from jax.experimental import pallas as pl
import jax.experimental.pallas.tpu as pltpu
from jax.experimental.pallas import tpu_sc as plsc

_TOP_K = 2
_ALPHA = 0.1
_E = 16
_ROW = 128


def _logits_body(hs_ref, w_ref, out_ref):
    out_ref[:, :_E] = jnp.dot(hs_ref[...], w_ref[...],
                              preferred_element_type=jnp.float32)


def _tc_logits(hs, weight):
    n_tok, h = hs.shape
    n_exp = weight.shape[0]
    block_t = 1024
    return pl.pallas_call(
        _logits_body,
        grid=(n_tok // block_t,),
        in_specs=[
            pl.BlockSpec((block_t, h), lambda i: (i, 0)),
            pl.BlockSpec((h, n_exp), lambda i: (0, 0)),
        ],
        out_specs=pl.BlockSpec((block_t, _ROW), lambda i: (i, 0)),
        out_shape=jax.ShapeDtypeStruct((n_tok, _ROW), jnp.float32),
    )(hs, weight.T)


def _sc_route(logits_p):
    n_tok = logits_p.shape[0]
    info = plsc.get_sparse_core_info()
    nw = info.num_cores * info.num_subcores
    t_per_w = n_tok // nw
    chunk = 128
    n_chunks = t_per_w // chunk

    mesh = plsc.VectorSubcoreMesh(core_axis_name="c", subcore_axis_name="s")

    @functools.partial(
        pl.kernel, mesh=mesh,
        compiler_params=pltpu.CompilerParams(needs_layout_passes=False),
        out_type=[
            jax.ShapeDtypeStruct((n_tok, _ROW), jnp.int32),
            jax.ShapeDtypeStruct((n_tok, _ROW), jnp.float32),
            jax.ShapeDtypeStruct((nw, _ROW), jnp.float32),
        ],
        scratch_types=[
            pltpu.VMEM((chunk, _ROW), jnp.float32),
            pltpu.VMEM((chunk, _ROW), jnp.int32),
            pltpu.VMEM((chunk, _ROW), jnp.float32),
            pltpu.VMEM((1, _ROW), jnp.float32),
        ],
    )
    def route(lg_hbm, idx_hbm, w_hbm, agg_hbm, lg_v, idx_v, w_v, agg_v):
        wid = lax.axis_index("s") * info.num_cores + lax.axis_index("c")
        base = wid * t_per_w

        iota = lax.broadcasted_iota(jnp.int32, (_E,), 0)
        zeros_i = iota * 0
        ones_i = zeros_i + 1
        dnums = lax.GatherDimensionNumbers(
            offset_dims=(), collapsed_slice_dims=(0,), start_index_map=(0,))

        def _take(x, i):
            return lax.gather(
                x, i[:, None], dnums, (1,),
                mode=lax.GatherScatterMode.PROMISE_IN_BOUNDS)

        def body(r, carry):
            acc_cnt, acc_ssum = carry
            lg = lg_v[r, pl.ds(0, _E)]
            srt_lg, srt_i = plsc.sort_key_val(lg, iota, descending=True)
            bc0 = _take(srt_lg, zeros_i)
            bc1 = _take(srt_lg, ones_i)
            e_v = jnp.exp(lg - bc0)
            z = e_v
            for k in (1, 2, 4, 8):
                z = z + _take(z, (iota + k) & (_E - 1))
            e2 = jnp.exp(bc1 - bc0)
            w_full = e_v / (1.0 + e2)
            srt_lg2, srt_w = plsc.sort_key_val(lg, w_full, descending=True)
            idx_v[r, pl.ds(0, _E)] = srt_i
            w_v[r, pl.ds(0, _E)] = srt_w
            oh1 = (lg == bc0).astype(jnp.float32)
            oh2 = (lg == bc1).astype(jnp.float32)
            return acc_cnt + oh1 + oh2, acc_ssum + e_v / z

        zero = jnp.zeros((_E,), jnp.float32)
        acc = (zero, zero)
        for c in range(n_chunks):
            off = base + c * chunk
            pltpu.sync_copy(lg_hbm.at[pl.ds(off, chunk)], lg_v)
            acc = lax.fori_loop(0, chunk, body, acc)
            pltpu.sync_copy(idx_v, idx_hbm.at[pl.ds(off, chunk)])
            pltpu.sync_copy(w_v, w_hbm.at[pl.ds(off, chunk)])

        agg_v[0, pl.ds(0, _E)] = acc[0]
        agg_v[0, pl.ds(_E, _E)] = acc[1]
        pltpu.sync_copy(agg_v, agg_hbm.at[pl.ds(wid, 1)])

    return route(logits_p)


def kernel(hidden_states, weight):
    bsz, seq_len, h = hidden_states.shape
    hs = hidden_states.reshape(-1, h)
    n_tok = hs.shape[0]

    logits_p = _tc_logits(hs, weight)
    idx_s, w_s, agg = _sc_route(logits_p)

    topk_idx = idx_s[:, :_TOP_K]
    topk_w = w_s[:, :_TOP_K]

    nw = agg.shape[0]
    w_per_b = nw // bsz
    cnt_b = agg[:, :_E].reshape(bsz, w_per_b, _E).sum(axis=1)
    ssum_b = agg[:, _E:2 * _E].reshape(bsz, w_per_b, _E).sum(axis=1)
    ce = cnt_b / (seq_len * _TOP_K / _E)
    smean = ssum_b / seq_len
    aux_loss = jnp.mean(jnp.sum(ce * smean, axis=1)) * _ALPHA
    return topk_idx, topk_w, aux_loss


# speedup vs baseline: 2.0989x; 2.0989x over previous
import functools

import jax
import jax.numpy as jnp
from jax.experimental import pallas as pl
import jax.experimental.pallas.tpu as pltpu

_TOP_K = 2
_ALPHA = 0.1


def _gate_body(seq_len, blocks_per_batch, n_exp, hs_ref, w_ref, idx_ref, tw_ref,
               aux_ref, ssum_ref, cnt_ref):
    i = pl.program_id(0)
    s = jax.lax.rem(i, blocks_per_batch)

    x = hs_ref[...]
    logits = jax.lax.dot_general(
        w_ref[...], x, (((1,), (1,)), ((), ())),
        preferred_element_type=jnp.float32)
    t = logits.shape[1]

    m1 = jnp.max(logits, axis=0, keepdims=True)
    e = jnp.exp(logits - m1)
    z = jnp.sum(e, axis=0, keepdims=True)
    scores = e / z

    iota = jax.lax.broadcasted_iota(jnp.int32, (n_exp, t), 0)
    a1 = jnp.min(jnp.where(logits == m1, iota, n_exp), axis=0, keepdims=True)
    oh1 = iota == a1
    masked = jnp.where(oh1, -jnp.inf, logits)
    m2 = jnp.max(masked, axis=0, keepdims=True)
    a2 = jnp.min(jnp.where(masked == m2, iota, n_exp), axis=0, keepdims=True)
    oh2 = iota == a2

    p1 = 1.0 / z
    p2 = jnp.exp(m2 - m1) / z
    denom = p1 + p2 + 1e-20
    idx_ref[...] = jnp.concatenate([a1, a2], axis=0)
    tw_ref[...] = jnp.concatenate([p1 / denom, p2 / denom], axis=0)

    blk_cnt = jnp.sum(oh1.astype(jnp.float32) + oh2.astype(jnp.float32),
                      axis=1, keepdims=True)
    blk_ssum = jnp.sum(scores, axis=1, keepdims=True)

    @pl.when(s == 0)
    def _init():
        cnt_ref[...] = blk_cnt
        ssum_ref[...] = blk_ssum

    @pl.when(s != 0)
    def _acc():
        cnt_ref[...] += blk_cnt
        ssum_ref[...] += blk_ssum

    @pl.when(i == 0)
    def _zero_aux():
        aux_ref[...] = jnp.zeros_like(aux_ref)

    @pl.when(s == blocks_per_batch - 1)
    def _finish_batch():
        ce = cnt_ref[...] / (seq_len * _TOP_K / n_exp)
        smean = ssum_ref[...] / seq_len
        aux_ref[...] += jnp.sum(ce * smean, axis=0, keepdims=True)


def kernel(hidden_states, weight):
    bsz, seq_len, h = hidden_states.shape
    n_exp = weight.shape[0]
    hs = hidden_states.reshape(-1, h)
    n_tok = hs.shape[0]

    block_t = 1024
    blocks_per_batch = seq_len // block_t
    grid = (n_tok // block_t,)

    body = functools.partial(_gate_body, seq_len, blocks_per_batch, n_exp)
    topk_idx, topk_w, aux = pl.pallas_call(
        body,
        grid=grid,
        in_specs=[
            pl.BlockSpec((block_t, h), lambda i: (i, 0)),
            pl.BlockSpec((n_exp, h), lambda i: (0, 0)),
        ],
        out_specs=[
            pl.BlockSpec((_TOP_K, block_t), lambda i: (0, i)),
            pl.BlockSpec((_TOP_K, block_t), lambda i: (0, i)),
            pl.BlockSpec((1, 1), lambda i: (0, 0)),
        ],
        out_shape=[
            jax.ShapeDtypeStruct((_TOP_K, n_tok), jnp.int32),
            jax.ShapeDtypeStruct((_TOP_K, n_tok), jnp.float32),
            jax.ShapeDtypeStruct((1, 1), jnp.float32),
        ],
        scratch_shapes=[
            pltpu.VMEM((n_exp, 1), jnp.float32),
            pltpu.VMEM((n_exp, 1), jnp.float32),
        ],
    )(hs, weight)

    aux_loss = aux[0, 0] * (_ALPHA / bsz)
    return topk_idx.T, topk_w.T, aux_loss
